# baseline (speedup 1.0000x reference)
import functools

import jax
import jax.numpy as jnp
from jax import lax
from jax.experimental import pallas as pl
from jax.experimental.pallas import tpu as pltpu
from jax.experimental.pallas import tpu_sc as plsc

_B, _T, _H = 64, 512, 768
_N = _B * _T
_EPS = 1e-12

_S = 1
_BS = _B // _S
_NS_TOK = _N // _S

_NC, _NSC = 2, 16
_NW = _NC * _NSC
_RPW = _NS_TOK // _NW
_CHUNK = 64
_NCHUNK = _RPW // _CHUNK


def _sc_gather(word_table, ids):
    mesh = plsc.VectorSubcoreMesh(
        core_axis_name="c", subcore_axis_name="s",
        num_cores=_NC, num_subcores=_NSC)

    @functools.partial(
        pl.kernel,
        out_type=jax.ShapeDtypeStruct((_NS_TOK, _H), jnp.float32),
        mesh=mesh,
        scratch_types=[
            pltpu.VMEM((_RPW,), jnp.int32),
            pltpu.VMEM((_CHUNK, _H), jnp.float32),
            pltpu.VMEM((_CHUNK, _H), jnp.float32),
            pltpu.SemaphoreType.DMA,
            pltpu.SemaphoreType.DMA,
            pltpu.SemaphoreType.DMA,
            pltpu.SemaphoreType.DMA,
        ],
    )
    def k(word_hbm, ids_hbm, out_hbm, idx_v, rows0, rows1, g0, g1, w0, w1):
        wid = lax.axis_index("s") * _NC + lax.axis_index("c")
        base = wid * _RPW
        rows = (rows0, rows1)
        gsem = (g0, g1)
        wsem = (w0, w1)
        pltpu.sync_copy(ids_hbm.at[pl.ds(base, _RPW)], idx_v)

        def gather(c):
            b = c % 2
            return pltpu.async_copy(
                word_hbm.at[idx_v.at[pl.ds(c * _CHUNK, _CHUNK)]],
                rows[b], gsem[b])

        writes = [None, None]
        gathers = [None] * _NCHUNK
        gathers[0] = gather(0)
        for c in range(_NCHUNK):
            b = c % 2
            if c + 1 < _NCHUNK:
                if writes[1 - b] is not None:
                    writes[1 - b].wait()
                gathers[c + 1] = gather(c + 1)
            gathers[c].wait()
            writes[b] = pltpu.async_copy(
                rows[b], out_hbm.at[pl.ds(base + c * _CHUNK, _CHUNK)],
                wsem[b])
        writes[0].wait()
        writes[1].wait()

    return k(word_table, ids)


def _tc_body(wemb_ref, tt_ids_ref, pos_ref, tt_ref, g_ref, b_ref, out_ref):
    x = wemb_ref[...]
    m = tt_ids_ref[...] == 1
    x = x + pos_ref[...] + jnp.where(m, tt_ref[1:2, :], tt_ref[0:1, :])
    mean = jnp.mean(x, axis=-1, keepdims=True)
    xc = x - mean
    var = jnp.mean(xc * xc, axis=-1, keepdims=True)
    y = xc * lax.rsqrt(var + _EPS)
    out_ref[...] = y * g_ref[...] + b_ref[...]


def _tc_body_acc(y_ref, wemb_ref, tt_ids_ref, pos_ref, tt_ref, g_ref, b_ref,
                 out_ref):
    del y_ref
    _tc_body(wemb_ref, tt_ids_ref, pos_ref, tt_ref, g_ref, b_ref, out_ref)


_DENSE_SPECS = [
    pl.BlockSpec((_T, _H), lambda i: (i, 0)),
    pl.BlockSpec((_T, 1), lambda i: (i, 0)),
    pl.BlockSpec((_T, _H), lambda i: (0, 0)),
    pl.BlockSpec((2, _H), lambda i: (0, 0)),
    pl.BlockSpec((1, _H), lambda i: (0, 0)),
    pl.BlockSpec((1, _H), lambda i: (0, 0)),
]


def _tc_add_ln_slice(y, wemb_s, tt_ids_s, pos, tt, g, b, s):
    out_spec = pl.BlockSpec((_T, _H), lambda i, s=s: (s * _BS + i, 0))
    if y is None:
        return pl.pallas_call(
            _tc_body,
            grid=(_BS,),
            in_specs=_DENSE_SPECS,
            out_specs=out_spec,
            out_shape=jax.ShapeDtypeStruct((_N, _H), jnp.float32),
        )(wemb_s, tt_ids_s, pos, tt, g, b)
    return pl.pallas_call(
        _tc_body_acc,
        grid=(_BS,),
        in_specs=[pl.BlockSpec(memory_space=pl.ANY)] + _DENSE_SPECS,
        out_specs=out_spec,
        out_shape=jax.ShapeDtypeStruct((_N, _H), jnp.float32),
        input_output_aliases={0: 0},
    )(y, wemb_s, tt_ids_s, pos, tt, g, b)


def kernel(input_ids, token_type_ids, word_table, pos_table, tt_table, gamma, beta):
    ids = input_ids.reshape(-1).astype(jnp.int32)
    tt_ids = token_type_ids.reshape(-1, 1).astype(jnp.int32)
    g = gamma.reshape(1, _H)
    b = beta.reshape(1, _H)

    wembs = [_sc_gather(word_table, ids[s * _NS_TOK:(s + 1) * _NS_TOK])
             for s in range(_S)]
    y = None
    for s in range(_S):
        tt_s = tt_ids[s * _NS_TOK:(s + 1) * _NS_TOK]
        y = _tc_add_ln_slice(y, wembs[s], tt_s, pos_table, tt_table, g, b, s)
    return y.reshape(_B, _T, _H)
